# baseline (speedup 1.0000x reference)
import jax
import jax.numpy as jnp
from jax import lax
from jax.experimental import pallas as pl
from jax.experimental.pallas import tpu as pltpu
from jax.experimental.pallas import tpu_sc as plsc

NDEV = 8
T = 4096
D = 1024
TOP_K = 2
NEXP = 16

NC = 2
NS = 16
NW = NC * NS
TPW = T // NW
ZROWS = 16
CHUNK = 16
NCHUNK = TPW // CHUNK


def _dispatch_body(in_hbm, idx_hbm, map_hbm, out_hbm,
                   zero_v, idx_v, map_v, gbuf, zsem, gsem):
    cid = lax.axis_index("c")
    sid = lax.axis_index("s")
    wid = sid * NC + cid
    base_t = wid * TPW

    def _zrow(i, carry):
        def _zcol(j, c2):
            zero_v[i, pl.ds(j * 16, 16)] = jnp.zeros((16,), jnp.float32)
            return c2
        return lax.fori_loop(0, D // 16, _zcol, carry)
    lax.fori_loop(0, ZROWS, _zrow, 0)

    pltpu.sync_copy(idx_hbm.at[pl.ds(wid * (TPW * TOP_K), TPW * TOP_K)], idx_v)
    pltpu.sync_copy(map_hbm, map_v)

    zhandles = []
    for d in range(NDEV):
        for c in range(TPW // ZROWS):
            dst = out_hbm.at[pl.ds(d * T + base_t + c * ZROWS, ZROWS)]
            zhandles.append(pltpu.async_copy(zero_v, dst, zsem))
    for h in zhandles:
        h.wait()

    io16 = lax.iota(jnp.int32, 16)
    for c in range(NCHUNK):
        t0 = base_t + c * CHUNK
        e0 = plsc.load_gather(idx_v, [c * (2 * CHUNK) + 2 * io16])
        e1 = plsc.load_gather(idx_v, [c * (2 * CHUNK) + 2 * io16 + 1])
        d0 = plsc.load_gather(map_v, [e0])
        d1 = plsc.load_gather(map_v, [e1])
        tv = t0 + io16
        dst0 = d0 * T + tv
        dst1 = d1 * T + tv
        pltpu.sync_copy(in_hbm.at[pl.ds(t0, CHUNK)], gbuf)
        h0 = pltpu.async_copy(gbuf, out_hbm.at[dst0], gsem)
        h1 = pltpu.async_copy(gbuf, out_hbm.at[dst1], gsem)
        h0.wait()
        h1.wait()


def kernel(input_tensor, expert_indices, expert_mapping):
    idx_flat = expert_indices.reshape(-1)
    mesh = plsc.VectorSubcoreMesh(
        core_axis_name="c", subcore_axis_name="s",
        num_cores=NC, num_subcores=NS)
    f = pl.kernel(
        _dispatch_body,
        out_type=jax.ShapeDtypeStruct((NDEV * T, D), jnp.float32),
        mesh=mesh,
        compiler_params=pltpu.CompilerParams(needs_layout_passes=False),
        scratch_types=[
            pltpu.VMEM((ZROWS, D), jnp.float32),
            pltpu.VMEM((TPW * TOP_K,), jnp.int32),
            pltpu.VMEM((NEXP,), jnp.int32),
            pltpu.VMEM((CHUNK, D), jnp.float32),
            pltpu.SemaphoreType.DMA,
            pltpu.SemaphoreType.DMA,
        ],
    )
    out = f(input_tensor, idx_flat, expert_mapping)
    return out.reshape(NDEV, T, D)


# speedup vs baseline: 1.1212x; 1.1212x over previous
import jax
import jax.numpy as jnp
from jax import lax
from jax.experimental import pallas as pl
from jax.experimental.pallas import tpu as pltpu
from jax.experimental.pallas import tpu_sc as plsc

NDEV = 8
T = 4096
D = 1024
TOP_K = 2
NEXP = 16

NC = 2
NS = 16
NW = NC * NS
TPW = T // NW
CHUNK = 16
NCHUNK = TPW // CHUNK


def _dispatch_body(in_hbm, idx_hbm, map_hbm, out_hbm,
                   zero_v, idx_v, map_v, gbufs, isem, zsems, gsems, ssems):
    cid = lax.axis_index("c")
    sid = lax.axis_index("s")
    wid = sid * NC + cid
    base_t = wid * TPW

    ih0 = pltpu.async_copy(
        idx_hbm.at[pl.ds(wid * (TPW * TOP_K), TPW * TOP_K)], idx_v, isem)
    ih1 = pltpu.async_copy(map_hbm, map_v, isem)

    ghandles = [None] * NCHUNK
    ghandles[0] = pltpu.async_copy(
        in_hbm.at[pl.ds(base_t, CHUNK)], gbufs[0], gsems[0])

    def _zrow(i, carry):
        for j in range(D // 16):
            zero_v[i, pl.ds(j * 16, 16)] = jnp.zeros((16,), jnp.float32)
        return carry
    lax.fori_loop(0, CHUNK, _zrow, 0)

    def fire_zeros(c):
        hs = []
        for d in range(NDEV):
            dst = out_hbm.at[pl.ds(d * T + base_t + c * CHUNK, CHUNK)]
            hs.append(pltpu.async_copy(zero_v, dst, zsems[c % 2]))
        return hs

    zhandles = [None] * NCHUNK
    zhandles[0] = fire_zeros(0)
    zhandles[1] = fire_zeros(1)

    ih0.wait()
    ih1.wait()

    io16 = lax.iota(jnp.int32, 16)
    shandles = [None] * NCHUNK
    for c in range(NCHUNK):
        if c >= 1:
            for h in shandles[c - 1]:
                h.wait()
        if c + 1 < NCHUNK:
            ghandles[c + 1] = pltpu.async_copy(
                in_hbm.at[pl.ds(base_t + (c + 1) * CHUNK, CHUNK)],
                gbufs[(c + 1) % 2], gsems[(c + 1) % 2])

        t0 = base_t + c * CHUNK
        e0 = plsc.load_gather(idx_v, [c * (2 * CHUNK) + 2 * io16])
        e1 = plsc.load_gather(idx_v, [c * (2 * CHUNK) + 2 * io16 + 1])
        d0 = plsc.load_gather(map_v, [e0])
        d1 = plsc.load_gather(map_v, [e1])
        tv = t0 + io16
        dst0 = d0 * T + tv
        dst1 = d1 * T + tv

        ghandles[c].wait()
        for h in zhandles[c]:
            h.wait()
        if c + 2 < NCHUNK:
            zhandles[c + 2] = fire_zeros(c + 2)

        shandles[c] = [
            pltpu.async_copy(gbufs[c % 2], out_hbm.at[dst0], ssems[c % 2]),
            pltpu.async_copy(gbufs[c % 2], out_hbm.at[dst1], ssems[c % 2]),
        ]

    for h in shandles[NCHUNK - 1]:
        h.wait()


def kernel(input_tensor, expert_indices, expert_mapping):
    idx_flat = expert_indices.reshape(-1)
    mesh = plsc.VectorSubcoreMesh(
        core_axis_name="c", subcore_axis_name="s",
        num_cores=NC, num_subcores=NS)
    f = pl.kernel(
        _dispatch_body,
        out_type=jax.ShapeDtypeStruct((NDEV * T, D), jnp.float32),
        mesh=mesh,
        compiler_params=pltpu.CompilerParams(needs_layout_passes=False),
        scratch_types=[
            pltpu.VMEM((CHUNK, D), jnp.float32),
            pltpu.VMEM((TPW * TOP_K,), jnp.int32),
            pltpu.VMEM((NEXP,), jnp.int32),
            [pltpu.VMEM((CHUNK, D), jnp.float32),
             pltpu.VMEM((CHUNK, D), jnp.float32)],
            pltpu.SemaphoreType.DMA,
            [pltpu.SemaphoreType.DMA, pltpu.SemaphoreType.DMA],
            [pltpu.SemaphoreType.DMA, pltpu.SemaphoreType.DMA],
            [pltpu.SemaphoreType.DMA, pltpu.SemaphoreType.DMA],
        ],
    )
    out = f(input_tensor, idx_flat, expert_mapping)
    return out.reshape(NDEV, T, D)
